# baseline (speedup 1.0000x reference)
import functools

import jax
import jax.numpy as jnp
from jax.experimental import pallas as pl
from jax.experimental.pallas import tpu as pltpu

_NPOINT = 1024
_NSAMPLE = 64
_EPS = 1e-3


def _fps_body(xs_ref, ys_ref, zs_ref, idx_ref, bx_ref, by_ref, bz_ref, *, nb):
    rows, cols = 8, xs_ref.shape[1]
    xs = [xs_ref[8 * b:8 * b + 8, :] for b in range(nb)]
    ys = [ys_ref[8 * b:8 * b + 8, :] for b in range(nb)]
    zs = [zs_ref[8 * b:8 * b + 8, :] for b in range(nb)]
    flat = (jax.lax.broadcasted_iota(jnp.int32, (rows, cols), 0) * cols
            + jax.lax.broadcasted_iota(jnp.int32, (rows, cols), 1))
    flat2 = (jax.lax.broadcasted_iota(jnp.int32, (8, 128), 0) * 128
             + jax.lax.broadcasted_iota(jnp.int32, (8, 128), 1))

    def step(t, carry):
        far, dist, ib, bx, by, bz = carry
        rec = flat2 == t
        out = ([], [], [], [], [], [])
        for b in range(nb):
            sel = flat == far[b]
            cx = jnp.sum(jnp.where(sel, xs[b], 0.0))
            cy = jnp.sum(jnp.where(sel, ys[b], 0.0))
            cz = jnp.sum(jnp.where(sel, zs[b], 0.0))
            out[2].append(jnp.where(rec, far[b], ib[b]))
            out[3].append(jnp.where(rec, cx, bx[b]))
            out[4].append(jnp.where(rec, cy, by[b]))
            out[5].append(jnp.where(rec, cz, bz[b]))
            dx = xs[b] - cx
            dy = ys[b] - cy
            dz = zs[b] - cz
            d = (dx * dx + dy * dy) + dz * dz
            nd = jnp.minimum(dist[b], d)
            out[1].append(nd)
            m = jnp.max(nd)
            out[0].append(jnp.min(jnp.where(nd == m, flat,
                                            jnp.int32(2 ** 30))))
        return out

    far0 = [jnp.int32(0)] * nb
    dist0 = [jnp.full((rows, cols), 1e10, dtype=jnp.float32)] * nb
    zi = [jnp.zeros((8, 128), jnp.int32)] * nb
    zf = [jnp.zeros((8, 128), jnp.float32)] * nb
    _, _, ib, bx, by, bz = jax.lax.fori_loop(
        0, _NPOINT, step, (far0, dist0, zi, list(zf), list(zf), list(zf)))
    for b in range(nb):
        idx_ref[8 * b:8 * b + 8, :] = ib[b]
        bx_ref[8 * b:8 * b + 8, :] = bx[b]
        by_ref[8 * b:8 * b + 8, :] = by[b]
        bz_ref[8 * b:8 * b + 8, :] = bz[b]


def _fps(xyz, npoint):
    B, N, _ = xyz.shape
    xs = xyz[:, :, 0].reshape(B * 8, N // 8)
    ys = xyz[:, :, 1].reshape(B * 8, N // 8)
    zs = xyz[:, :, 2].reshape(B * 8, N // 8)
    idx, bx, by, bz = pl.pallas_call(
        functools.partial(_fps_body, nb=B),
        grid=(1,),
        in_specs=[pl.BlockSpec((B * 8, N // 8), lambda i: (0, 0))] * 3,
        out_specs=[pl.BlockSpec((B * 8, 128), lambda i: (0, 0))] * 4,
        out_shape=[
            jax.ShapeDtypeStruct((B * 8, 128), jnp.int32),
            jax.ShapeDtypeStruct((B * 8, 128), jnp.float32),
            jax.ShapeDtypeStruct((B * 8, 128), jnp.float32),
            jax.ShapeDtypeStruct((B * 8, 128), jnp.float32),
        ],
    )(xs, ys, zs)
    new_xyz = jnp.stack(
        [bx.reshape(B, npoint), by.reshape(B, npoint), bz.reshape(B, npoint)],
        axis=-1)
    return idx.reshape(B, npoint), new_xyz


def _knn_body(xs_ref, ys_ref, zs_ref, qx_ref, qy_ref, qz_ref, out_ref, d_ref,
              *, n, k, qt):
    xs = xs_ref[...].reshape(1, n)
    ys = ys_ref[...].reshape(1, n)
    zs = zs_ref[...].reshape(1, n)
    dx = qx_ref[...].reshape(qt, 1) - xs
    dy = qy_ref[...].reshape(qt, 1) - ys
    dz = qz_ref[...].reshape(qt, 1) - zs
    d_ref[...] = (dx * dx + dy * dy) + dz * dz
    li = jax.lax.broadcasted_iota(jnp.int32, (qt, n), 1)
    cols = jax.lax.broadcasted_iota(jnp.int32, (qt, k), 1)

    def it(jj, carry):
        buf, prev = carry
        dm = jnp.where(li == prev, jnp.float32(jnp.inf), d_ref[...])
        d_ref[...] = dm
        am = jnp.argmin(dm, axis=1).astype(jnp.int32).reshape(qt, 1)
        return jnp.where(cols == jj, am, buf), am

    buf, _ = jax.lax.fori_loop(
        0, k, it,
        (jnp.zeros((qt, k), jnp.int32), jnp.full((qt, 1), -1, jnp.int32)))
    out_ref[...] = buf


def _knn(k, xyz, new_xyz):
    B, N, _ = xyz.shape
    M = new_xyz.shape[1]
    QT = 128
    xs3 = xyz[:, :, 0].reshape(B, 1, N)
    ys3 = xyz[:, :, 1].reshape(B, 1, N)
    zs3 = xyz[:, :, 2].reshape(B, 1, N)
    jt = M // QT

    def qv(c):
        return c.reshape(B * jt, QT, 1)

    qx = qv(new_xyz[:, :, 0])
    qy = qv(new_xyz[:, :, 1])
    qz = qv(new_xyz[:, :, 2])
    idx = pl.pallas_call(
        functools.partial(_knn_body, n=N, k=k, qt=QT),
        grid=(B, jt),
        in_specs=[pl.BlockSpec((1, 1, N), lambda b, j: (b, 0, 0))] * 3
        + [pl.BlockSpec((1, QT, 1), lambda b, j: (b * jt + j, 0, 0))] * 3,
        out_specs=pl.BlockSpec((QT, k), lambda b, j: (b * jt + j, 0)),
        out_shape=jax.ShapeDtypeStruct((B * M, k), jnp.int32),
        scratch_shapes=[pltpu.VMEM((QT, N), jnp.float32)],
    )(xs3, ys3, zs3, qx, qy, qz)
    return idx.reshape(B, M, k)


def _sc_gather_body(tbl_ref, gidx_ref, out_ref, idx_v, rows_v, sem):
    wid = jax.lax.axis_index("s") * 2 + jax.lax.axis_index("c")
    per_w = idx_v.shape[0]
    nch = per_w // 128
    base = wid * per_w
    pltpu.sync_copy(gidx_ref.at[pl.ds(base, per_w)], idx_v)

    def chunk(c, _):
        pltpu.async_copy(tbl_ref.at[idx_v.at[pl.ds(c * 128, 128)]],
                         rows_v, sem).wait()
        pltpu.sync_copy(rows_v, out_ref.at[pl.ds(base + c * 128, 128)])
        return 0

    jax.lax.fori_loop(0, nch, chunk, 0)


def _sc_gather(tbl, gidx):
    from jax.experimental.pallas import tpu_sc as plsc
    R = gidx.shape[0]
    D = tbl.shape[1]
    per_w = R // 32
    mesh = plsc.VectorSubcoreMesh(core_axis_name="c", subcore_axis_name="s")
    f = pl.kernel(
        _sc_gather_body,
        mesh=mesh,
        out_type=jax.ShapeDtypeStruct((R, D), jnp.float32),
        scratch_types=[
            pltpu.VMEM((per_w,), jnp.int32),
            pltpu.VMEM((128, D), jnp.float32),
            pltpu.SemaphoreType.DMA,
        ],
    )
    return f(tbl, gidx)


def _pass_first(x_ref, w_ref, qx_ref, wq_ref, b_ref, y_ref, st_ref, *, bq):
    y = jnp.dot(x_ref[...], w_ref[...], preferred_element_type=jnp.float32)
    qc = b_ref[...] - jnp.dot(qx_ref[...], wq_ref[...],
                              preferred_element_type=jnp.float32)
    c = qc.shape[-1]
    qce = jnp.broadcast_to(qc.reshape(bq, 1, c),
                           (bq, _NSAMPLE, c)).reshape(bq * _NSAMPLE, c)
    y = y + qce
    y_ref[...] = y

    @pl.when(pl.program_id(0) == 0)
    def _():
        st_ref[...] = jnp.zeros_like(st_ref)

    st_ref[0:1, :] += jnp.sum(y, axis=0, keepdims=True)
    st_ref[1:2, :] += jnp.sum(y * y, axis=0, keepdims=True)


def _pass_mid(x_ref, w_ref, b_ref, sc_ref, sh_ref, y_ref, st_ref):
    h = jnp.maximum(x_ref[...] * sc_ref[...] + sh_ref[...], 0.0)
    y = jnp.dot(h, w_ref[...], preferred_element_type=jnp.float32)
    y = y + b_ref[...]
    y_ref[...] = y

    @pl.when(pl.program_id(0) == 0)
    def _():
        st_ref[...] = jnp.zeros_like(st_ref)

    st_ref[0:1, :] += jnp.sum(y, axis=0, keepdims=True)
    st_ref[1:2, :] += jnp.sum(y * y, axis=0, keepdims=True)


def _pass_last(x_ref, w_ref, b_ref, sc_ref, sh_ref, mx_ref, mn_ref, st_ref, *, bq):
    h = jnp.maximum(x_ref[...] * sc_ref[...] + sh_ref[...], 0.0)
    y = jnp.dot(h, w_ref[...], preferred_element_type=jnp.float32)
    y = y + b_ref[...]

    @pl.when(pl.program_id(0) == 0)
    def _():
        st_ref[...] = jnp.zeros_like(st_ref)

    st_ref[0:1, :] += jnp.sum(y, axis=0, keepdims=True)
    st_ref[1:2, :] += jnp.sum(y * y, axis=0, keepdims=True)
    for q in range(bq):
        blk = y[q * _NSAMPLE:(q + 1) * _NSAMPLE, :]
        mx_ref[q:q + 1, :] = jnp.max(blk, axis=0, keepdims=True)
        mn_ref[q:q + 1, :] = jnp.min(blk, axis=0, keepdims=True)


def _pass_pool(mx_ref, mn_ref, sc_ref, sh_ref, o_ref):
    sc = sc_ref[...]
    z = jnp.where(sc >= 0.0, sc * mx_ref[...], sc * mn_ref[...]) + sh_ref[...]
    o_ref[...] = jnp.maximum(z, 0.0)


def _moments_to_affine(st, n, gamma, beta):
    mean = st[0] / n
    var = st[1] / n - mean * mean
    rstd = jax.lax.rsqrt(var + _EPS)
    sc = gamma * rstd
    sh = beta - mean * sc
    return sc.reshape(1, -1), sh.reshape(1, -1)


def _conv_bn_pool(x, qx, W0, wq, b0, gamma0, beta0, W1, b1, gamma1, beta1,
                  W2, b2, gamma2, beta2):
    R, D = x.shape
    BR = 2048
    bq = BR // _NSAMPLE
    grid = R // BR
    n = jnp.float32(R)

    c0 = W0.shape[1]
    y1, st1 = pl.pallas_call(
        functools.partial(_pass_first, bq=bq),
        grid=(grid,),
        in_specs=[
            pl.BlockSpec((BR, D), lambda i: (i, 0)),
            pl.BlockSpec((D, c0), lambda i: (0, 0)),
            pl.BlockSpec((bq, 128), lambda i: (i, 0)),
            pl.BlockSpec((128, c0), lambda i: (0, 0)),
            pl.BlockSpec((1, c0), lambda i: (0, 0)),
        ],
        out_specs=[
            pl.BlockSpec((BR, c0), lambda i: (i, 0)),
            pl.BlockSpec((8, c0), lambda i: (0, 0)),
        ],
        out_shape=[
            jax.ShapeDtypeStruct((R, c0), jnp.float32),
            jax.ShapeDtypeStruct((8, c0), jnp.float32),
        ],
    )(x, W0, qx, wq, b0.reshape(1, -1))
    sc1, sh1 = _moments_to_affine(st1, n, gamma0, beta0)

    c1 = W1.shape[1]
    y2, st2 = pl.pallas_call(
        _pass_mid,
        grid=(grid,),
        in_specs=[
            pl.BlockSpec((BR, c0), lambda i: (i, 0)),
            pl.BlockSpec((c0, c1), lambda i: (0, 0)),
            pl.BlockSpec((1, c1), lambda i: (0, 0)),
            pl.BlockSpec((1, c0), lambda i: (0, 0)),
            pl.BlockSpec((1, c0), lambda i: (0, 0)),
        ],
        out_specs=[
            pl.BlockSpec((BR, c1), lambda i: (i, 0)),
            pl.BlockSpec((8, c1), lambda i: (0, 0)),
        ],
        out_shape=[
            jax.ShapeDtypeStruct((R, c1), jnp.float32),
            jax.ShapeDtypeStruct((8, c1), jnp.float32),
        ],
    )(y1, W1, b1.reshape(1, -1), sc1, sh1)
    sc2, sh2 = _moments_to_affine(st2, n, gamma1, beta1)

    c2 = W2.shape[1]
    bq = BR // _NSAMPLE
    Q = R // _NSAMPLE
    mx, mn, st3 = pl.pallas_call(
        functools.partial(_pass_last, bq=bq),
        grid=(grid,),
        in_specs=[
            pl.BlockSpec((BR, c1), lambda i: (i, 0)),
            pl.BlockSpec((c1, c2), lambda i: (0, 0)),
            pl.BlockSpec((1, c2), lambda i: (0, 0)),
            pl.BlockSpec((1, c1), lambda i: (0, 0)),
            pl.BlockSpec((1, c1), lambda i: (0, 0)),
        ],
        out_specs=[
            pl.BlockSpec((bq, c2), lambda i: (i, 0)),
            pl.BlockSpec((bq, c2), lambda i: (i, 0)),
            pl.BlockSpec((8, c2), lambda i: (0, 0)),
        ],
        out_shape=[
            jax.ShapeDtypeStruct((Q, c2), jnp.float32),
            jax.ShapeDtypeStruct((Q, c2), jnp.float32),
            jax.ShapeDtypeStruct((8, c2), jnp.float32),
        ],
    )(y2, W2, b2.reshape(1, -1), sc2, sh2)
    sc3, sh3 = _moments_to_affine(st3, n, gamma2, beta2)

    out = pl.pallas_call(
        _pass_pool,
        grid=(1,),
        in_specs=[
            pl.BlockSpec((Q, c2), lambda i: (0, 0)),
            pl.BlockSpec((Q, c2), lambda i: (0, 0)),
            pl.BlockSpec((1, c2), lambda i: (0, 0)),
            pl.BlockSpec((1, c2), lambda i: (0, 0)),
        ],
        out_specs=pl.BlockSpec((Q, c2), lambda i: (0, 0)),
        out_shape=jax.ShapeDtypeStruct((Q, c2), jnp.float32),
    )(mx, mn, sc3, sh3)
    return out


def kernel(xyz, points, W0, b0, gamma0, beta0, W1, b1, gamma1, beta1,
           W2, b2, gamma2, beta2):
    B, N, C = points.shape
    fps_idx, new_xyz = _fps(jax.lax.stop_gradient(xyz), _NPOINT)
    idx = _knn(_NSAMPLE, jax.lax.stop_gradient(xyz),
               jax.lax.stop_gradient(new_xyz))
    cin = 3 + C
    D = 128
    tbl = jnp.concatenate([xyz, points], axis=-1).reshape(B * N, cin)
    tbl = jnp.pad(tbl, ((0, 0), (0, D - cin)))
    gidx = (idx.reshape(B, -1)
            + (jnp.arange(B, dtype=jnp.int32) * N)[:, None]).reshape(-1)
    x = _sc_gather(tbl, gidx)
    W0p = jnp.pad(W0, ((0, D - cin), (0, 0)))
    wq = jnp.pad(W0[:3], ((0, 125), (0, 0)))
    qx = jnp.pad(new_xyz.reshape(B * _NPOINT, 3), ((0, 0), (0, 125)))

    pooled = _conv_bn_pool(x, qx, W0p, wq, b0, gamma0, beta0,
                           W1, b1, gamma1, beta1, W2, b2, gamma2, beta2)
    new_points = pooled.reshape(B, _NPOINT, W2.shape[1])
    return new_xyz, new_points, idx


# speedup vs baseline: 1.0609x; 1.0609x over previous
import functools

import jax
import jax.numpy as jnp
from jax.experimental import pallas as pl
from jax.experimental.pallas import tpu as pltpu

_NPOINT = 1024
_NSAMPLE = 64
_EPS = 1e-3


def _fps_body(xs_ref, ys_ref, zs_ref, idx_ref, bx_ref, by_ref, bz_ref, *, nb):
    rows, cols = 8, xs_ref.shape[1]
    xs = [xs_ref[8 * b:8 * b + 8, :] for b in range(nb)]
    ys = [ys_ref[8 * b:8 * b + 8, :] for b in range(nb)]
    zs = [zs_ref[8 * b:8 * b + 8, :] for b in range(nb)]
    flat = (jax.lax.broadcasted_iota(jnp.int32, (rows, cols), 0) * cols
            + jax.lax.broadcasted_iota(jnp.int32, (rows, cols), 1))
    flat2 = (jax.lax.broadcasted_iota(jnp.int32, (8, 128), 0) * 128
             + jax.lax.broadcasted_iota(jnp.int32, (8, 128), 1))

    def step(t, carry):
        far, dist, ib, bx, by, bz = carry
        rec = flat2 == t
        out = ([], [], [], [], [], [])
        for b in range(nb):
            sel = flat == far[b]
            cx = jnp.sum(jnp.where(sel, xs[b], 0.0))
            cy = jnp.sum(jnp.where(sel, ys[b], 0.0))
            cz = jnp.sum(jnp.where(sel, zs[b], 0.0))
            out[2].append(jnp.where(rec, far[b], ib[b]))
            out[3].append(jnp.where(rec, cx, bx[b]))
            out[4].append(jnp.where(rec, cy, by[b]))
            out[5].append(jnp.where(rec, cz, bz[b]))
            dx = xs[b] - cx
            dy = ys[b] - cy
            dz = zs[b] - cz
            d = (dx * dx + dy * dy) + dz * dz
            nd = jnp.minimum(dist[b], d)
            out[1].append(nd)
            m = jnp.max(nd)
            out[0].append(jnp.min(jnp.where(nd == m, flat,
                                            jnp.int32(2 ** 30))))
        return out

    far0 = [jnp.int32(0)] * nb
    dist0 = [jnp.full((rows, cols), 1e10, dtype=jnp.float32)] * nb
    zi = [jnp.zeros((8, 128), jnp.int32)] * nb
    zf = [jnp.zeros((8, 128), jnp.float32)] * nb
    _, _, ib, bx, by, bz = jax.lax.fori_loop(
        0, _NPOINT, step, (far0, dist0, zi, list(zf), list(zf), list(zf)))
    for b in range(nb):
        idx_ref[8 * b:8 * b + 8, :] = ib[b]
        bx_ref[8 * b:8 * b + 8, :] = bx[b]
        by_ref[8 * b:8 * b + 8, :] = by[b]
        bz_ref[8 * b:8 * b + 8, :] = bz[b]


def _fps(xyz, npoint):
    B, N, _ = xyz.shape
    xs = xyz[:, :, 0].reshape(B * 8, N // 8)
    ys = xyz[:, :, 1].reshape(B * 8, N // 8)
    zs = xyz[:, :, 2].reshape(B * 8, N // 8)
    idx, bx, by, bz = pl.pallas_call(
        functools.partial(_fps_body, nb=B),
        grid=(1,),
        in_specs=[pl.BlockSpec((B * 8, N // 8), lambda i: (0, 0))] * 3,
        out_specs=[pl.BlockSpec((B * 8, 128), lambda i: (0, 0))] * 4,
        out_shape=[
            jax.ShapeDtypeStruct((B * 8, 128), jnp.int32),
            jax.ShapeDtypeStruct((B * 8, 128), jnp.float32),
            jax.ShapeDtypeStruct((B * 8, 128), jnp.float32),
            jax.ShapeDtypeStruct((B * 8, 128), jnp.float32),
        ],
    )(xs, ys, zs)
    new_xyz = jnp.stack(
        [bx.reshape(B, npoint), by.reshape(B, npoint), bz.reshape(B, npoint)],
        axis=-1)
    return idx.reshape(B, npoint), new_xyz


def _knn_body(xs_ref, ys_ref, zs_ref, qx_ref, qy_ref, qz_ref, out_ref, d_ref,
              *, n, k, qt):
    xs = xs_ref[...].reshape(1, n)
    ys = ys_ref[...].reshape(1, n)
    zs = zs_ref[...].reshape(1, n)
    dx = qx_ref[...].reshape(qt, 1) - xs
    dy = qy_ref[...].reshape(qt, 1) - ys
    dz = qz_ref[...].reshape(qt, 1) - zs
    d_ref[...] = (dx * dx + dy * dy) + dz * dz
    li = jax.lax.broadcasted_iota(jnp.int32, (qt, n), 1)
    cols = jax.lax.broadcasted_iota(jnp.int32, (qt, k), 1)

    def it(jj, buf):
        dc = d_ref[...]
        am = jnp.argmin(dc, axis=1).astype(jnp.int32).reshape(qt, 1)
        d_ref[...] = jnp.where(li == am, jnp.float32(jnp.inf), dc)
        return jnp.where(cols == jj, am, buf)

    out_ref[...] = jax.lax.fori_loop(0, k, it, jnp.zeros((qt, k), jnp.int32))


def _knn(k, xyz, new_xyz):
    B, N, _ = xyz.shape
    M = new_xyz.shape[1]
    QT = 256
    xs3 = xyz[:, :, 0].reshape(B, 1, N)
    ys3 = xyz[:, :, 1].reshape(B, 1, N)
    zs3 = xyz[:, :, 2].reshape(B, 1, N)
    jt = M // QT

    def qv(c):
        return c.reshape(B * jt, QT, 1)

    qx = qv(new_xyz[:, :, 0])
    qy = qv(new_xyz[:, :, 1])
    qz = qv(new_xyz[:, :, 2])
    idx = pl.pallas_call(
        functools.partial(_knn_body, n=N, k=k, qt=QT),
        grid=(B, jt),
        in_specs=[pl.BlockSpec((1, 1, N), lambda b, j: (b, 0, 0))] * 3
        + [pl.BlockSpec((1, QT, 1), lambda b, j: (b * jt + j, 0, 0))] * 3,
        out_specs=pl.BlockSpec((QT, k), lambda b, j: (b * jt + j, 0)),
        out_shape=jax.ShapeDtypeStruct((B * M, k), jnp.int32),
        scratch_shapes=[pltpu.VMEM((QT, N), jnp.float32)],
    )(xs3, ys3, zs3, qx, qy, qz)
    return idx.reshape(B, M, k)


def _sc_gather_body(tbl_ref, gidx_ref, out_ref, idx_v, rows_v, sem):
    wid = jax.lax.axis_index("s") * 2 + jax.lax.axis_index("c")
    per_w = idx_v.shape[0]
    nch = per_w // 128
    base = wid * per_w
    pltpu.sync_copy(gidx_ref.at[pl.ds(base, per_w)], idx_v)

    def chunk(c, _):
        pltpu.async_copy(tbl_ref.at[idx_v.at[pl.ds(c * 128, 128)]],
                         rows_v, sem).wait()
        pltpu.sync_copy(rows_v, out_ref.at[pl.ds(base + c * 128, 128)])
        return 0

    jax.lax.fori_loop(0, nch, chunk, 0)


def _sc_gather(tbl, gidx):
    from jax.experimental.pallas import tpu_sc as plsc
    R = gidx.shape[0]
    D = tbl.shape[1]
    per_w = R // 32
    mesh = plsc.VectorSubcoreMesh(core_axis_name="c", subcore_axis_name="s")
    f = pl.kernel(
        _sc_gather_body,
        mesh=mesh,
        out_type=jax.ShapeDtypeStruct((R, D), jnp.float32),
        scratch_types=[
            pltpu.VMEM((per_w,), jnp.int32),
            pltpu.VMEM((128, D), jnp.float32),
            pltpu.SemaphoreType.DMA,
        ],
    )
    return f(tbl, gidx)


def _pass_first(x_ref, w_ref, qx_ref, wq_ref, b_ref, y_ref, st_ref, *, bq):
    y = jnp.dot(x_ref[...], w_ref[...], preferred_element_type=jnp.float32)
    qc = b_ref[...] - jnp.dot(qx_ref[...], wq_ref[...],
                              preferred_element_type=jnp.float32)
    c = qc.shape[-1]
    qce = jnp.broadcast_to(qc.reshape(bq, 1, c),
                           (bq, _NSAMPLE, c)).reshape(bq * _NSAMPLE, c)
    y = y + qce
    y_ref[...] = y

    @pl.when(pl.program_id(0) == 0)
    def _():
        st_ref[...] = jnp.zeros_like(st_ref)

    st_ref[0:1, :] += jnp.sum(y, axis=0, keepdims=True)
    st_ref[1:2, :] += jnp.sum(y * y, axis=0, keepdims=True)


def _pass_mid(x_ref, w_ref, b_ref, sc_ref, sh_ref, y_ref, st_ref):
    h = jnp.maximum(x_ref[...] * sc_ref[...] + sh_ref[...], 0.0)
    y = jnp.dot(h, w_ref[...], preferred_element_type=jnp.float32)
    y = y + b_ref[...]
    y_ref[...] = y

    @pl.when(pl.program_id(0) == 0)
    def _():
        st_ref[...] = jnp.zeros_like(st_ref)

    st_ref[0:1, :] += jnp.sum(y, axis=0, keepdims=True)
    st_ref[1:2, :] += jnp.sum(y * y, axis=0, keepdims=True)


def _pass_last(x_ref, w_ref, b_ref, sc_ref, sh_ref, mx_ref, mn_ref, st_ref, *, bq):
    h = jnp.maximum(x_ref[...] * sc_ref[...] + sh_ref[...], 0.0)
    y = jnp.dot(h, w_ref[...], preferred_element_type=jnp.float32)
    y = y + b_ref[...]

    @pl.when(pl.program_id(0) == 0)
    def _():
        st_ref[...] = jnp.zeros_like(st_ref)

    st_ref[0:1, :] += jnp.sum(y, axis=0, keepdims=True)
    st_ref[1:2, :] += jnp.sum(y * y, axis=0, keepdims=True)
    for q in range(bq):
        blk = y[q * _NSAMPLE:(q + 1) * _NSAMPLE, :]
        mx_ref[q:q + 1, :] = jnp.max(blk, axis=0, keepdims=True)
        mn_ref[q:q + 1, :] = jnp.min(blk, axis=0, keepdims=True)


def _pass_pool(mx_ref, mn_ref, sc_ref, sh_ref, o_ref):
    sc = sc_ref[...]
    z = jnp.where(sc >= 0.0, sc * mx_ref[...], sc * mn_ref[...]) + sh_ref[...]
    o_ref[...] = jnp.maximum(z, 0.0)


def _moments_to_affine(st, n, gamma, beta):
    mean = st[0] / n
    var = st[1] / n - mean * mean
    rstd = jax.lax.rsqrt(var + _EPS)
    sc = gamma * rstd
    sh = beta - mean * sc
    return sc.reshape(1, -1), sh.reshape(1, -1)


def _conv_bn_pool(x, qx, W0, wq, b0, gamma0, beta0, W1, b1, gamma1, beta1,
                  W2, b2, gamma2, beta2):
    R, D = x.shape
    BR = 2048
    bq = BR // _NSAMPLE
    grid = R // BR
    n = jnp.float32(R)

    c0 = W0.shape[1]
    y1, st1 = pl.pallas_call(
        functools.partial(_pass_first, bq=bq),
        grid=(grid,),
        in_specs=[
            pl.BlockSpec((BR, D), lambda i: (i, 0)),
            pl.BlockSpec((D, c0), lambda i: (0, 0)),
            pl.BlockSpec((bq, 128), lambda i: (i, 0)),
            pl.BlockSpec((128, c0), lambda i: (0, 0)),
            pl.BlockSpec((1, c0), lambda i: (0, 0)),
        ],
        out_specs=[
            pl.BlockSpec((BR, c0), lambda i: (i, 0)),
            pl.BlockSpec((8, c0), lambda i: (0, 0)),
        ],
        out_shape=[
            jax.ShapeDtypeStruct((R, c0), jnp.float32),
            jax.ShapeDtypeStruct((8, c0), jnp.float32),
        ],
    )(x, W0, qx, wq, b0.reshape(1, -1))
    sc1, sh1 = _moments_to_affine(st1, n, gamma0, beta0)

    c1 = W1.shape[1]
    y2, st2 = pl.pallas_call(
        _pass_mid,
        grid=(grid,),
        in_specs=[
            pl.BlockSpec((BR, c0), lambda i: (i, 0)),
            pl.BlockSpec((c0, c1), lambda i: (0, 0)),
            pl.BlockSpec((1, c1), lambda i: (0, 0)),
            pl.BlockSpec((1, c0), lambda i: (0, 0)),
            pl.BlockSpec((1, c0), lambda i: (0, 0)),
        ],
        out_specs=[
            pl.BlockSpec((BR, c1), lambda i: (i, 0)),
            pl.BlockSpec((8, c1), lambda i: (0, 0)),
        ],
        out_shape=[
            jax.ShapeDtypeStruct((R, c1), jnp.float32),
            jax.ShapeDtypeStruct((8, c1), jnp.float32),
        ],
    )(y1, W1, b1.reshape(1, -1), sc1, sh1)
    sc2, sh2 = _moments_to_affine(st2, n, gamma1, beta1)

    c2 = W2.shape[1]
    bq = BR // _NSAMPLE
    Q = R // _NSAMPLE
    mx, mn, st3 = pl.pallas_call(
        functools.partial(_pass_last, bq=bq),
        grid=(grid,),
        in_specs=[
            pl.BlockSpec((BR, c1), lambda i: (i, 0)),
            pl.BlockSpec((c1, c2), lambda i: (0, 0)),
            pl.BlockSpec((1, c2), lambda i: (0, 0)),
            pl.BlockSpec((1, c1), lambda i: (0, 0)),
            pl.BlockSpec((1, c1), lambda i: (0, 0)),
        ],
        out_specs=[
            pl.BlockSpec((bq, c2), lambda i: (i, 0)),
            pl.BlockSpec((bq, c2), lambda i: (i, 0)),
            pl.BlockSpec((8, c2), lambda i: (0, 0)),
        ],
        out_shape=[
            jax.ShapeDtypeStruct((Q, c2), jnp.float32),
            jax.ShapeDtypeStruct((Q, c2), jnp.float32),
            jax.ShapeDtypeStruct((8, c2), jnp.float32),
        ],
    )(y2, W2, b2.reshape(1, -1), sc2, sh2)
    sc3, sh3 = _moments_to_affine(st3, n, gamma2, beta2)

    out = pl.pallas_call(
        _pass_pool,
        grid=(1,),
        in_specs=[
            pl.BlockSpec((Q, c2), lambda i: (0, 0)),
            pl.BlockSpec((Q, c2), lambda i: (0, 0)),
            pl.BlockSpec((1, c2), lambda i: (0, 0)),
            pl.BlockSpec((1, c2), lambda i: (0, 0)),
        ],
        out_specs=pl.BlockSpec((Q, c2), lambda i: (0, 0)),
        out_shape=jax.ShapeDtypeStruct((Q, c2), jnp.float32),
    )(mx, mn, sc3, sh3)
    return out


def kernel(xyz, points, W0, b0, gamma0, beta0, W1, b1, gamma1, beta1,
           W2, b2, gamma2, beta2):
    B, N, C = points.shape
    fps_idx, new_xyz = _fps(jax.lax.stop_gradient(xyz), _NPOINT)
    idx = _knn(_NSAMPLE, jax.lax.stop_gradient(xyz),
               jax.lax.stop_gradient(new_xyz))
    cin = 3 + C
    D = 128
    tbl = jnp.concatenate([xyz, points], axis=-1).reshape(B * N, cin)
    tbl = jnp.pad(tbl, ((0, 0), (0, D - cin)))
    gidx = (idx.reshape(B, -1)
            + (jnp.arange(B, dtype=jnp.int32) * N)[:, None]).reshape(-1)
    x = _sc_gather(tbl, gidx)
    W0p = jnp.pad(W0, ((0, D - cin), (0, 0)))
    wq = jnp.pad(W0[:3], ((0, 125), (0, 0)))
    qx = jnp.pad(new_xyz.reshape(B * _NPOINT, 3), ((0, 0), (0, 125)))

    pooled = _conv_bn_pool(x, qx, W0p, wq, b0, gamma0, beta0,
                           W1, b1, gamma1, beta1, W2, b2, gamma2, beta2)
    new_points = pooled.reshape(B, _NPOINT, W2.shape[1])
    return new_xyz, new_points, idx
